# baseline (speedup 1.0000x reference)
import jax
import jax.numpy as jnp
from jax import lax
from jax.experimental import pallas as pl
from jax.experimental.pallas import tpu as pltpu
from jax.experimental.pallas import tpu_sc as plsc

BATCH = 16384
EMB = 64
NC = 2
NS = 16
NW = NC * NS
B_PER = BATCH // NW
CHUNK = 128
NCHUNK = B_PER // CHUNK


def _body(u_hbm, v_hbm, uemb_hbm, iemb_hbm, out_hbm,
          uidx, vidx, urows, vrows, outv, sem):
    wid = lax.axis_index("s") * NC + lax.axis_index("c")
    base = wid * B_PER

    pltpu.sync_copy(u_hbm.at[pl.ds(base, B_PER)], uidx)
    pltpu.sync_copy(v_hbm.at[pl.ds(base, B_PER)], vidx)

    copies = []
    for j in range(NCHUNK):
        sl = pl.ds(j * CHUNK, CHUNK)
        dst = pl.ds(j * CHUNK, CHUNK)
        copies.append(pltpu.async_copy(uemb_hbm.at[uidx.at[sl]], urows.at[dst], sem))
        copies.append(pltpu.async_copy(iemb_hbm.at[vidx.at[sl]], vrows.at[dst], sem))
    for c in copies:
        c.wait()

    lane = lax.broadcasted_iota(jnp.int32, (16,), 0)

    def group(g, _):
        row16 = g * 16 + lane
        acc = jnp.zeros((16,), jnp.float32)
        for d in range(EMB):
            col = jnp.full((16,), d, jnp.int32)
            cu = plsc.load_gather(urows, [row16, col])
            cv = plsc.load_gather(vrows, [row16, col])
            acc += cu * cv
        outv[pl.ds(g * 16, 16)] = acc
        return 0

    lax.fori_loop(0, B_PER // 16, group, 0)

    pltpu.sync_copy(outv, out_hbm.at[pl.ds(base, B_PER)])


@jax.jit
def kernel(u, v, user_emb, item_emb):
    mesh = plsc.VectorSubcoreMesh(core_axis_name="c", subcore_axis_name="s")
    k = pl.kernel(
        _body,
        out_type=jax.ShapeDtypeStruct((BATCH,), jnp.float32),
        mesh=mesh,
        compiler_params=pltpu.CompilerParams(
            needs_layout_passes=False, use_tc_tiling_on_sc=False),
        scratch_types=[
            pltpu.VMEM((B_PER,), jnp.int32),
            pltpu.VMEM((B_PER,), jnp.int32),
            pltpu.VMEM((B_PER, EMB), jnp.float32),
            pltpu.VMEM((B_PER, EMB), jnp.float32),
            pltpu.VMEM((B_PER,), jnp.float32),
            pltpu.SemaphoreType.DMA,
        ],
    )
    return k(u, v, user_emb, item_emb)
